# baseline (speedup 1.0000x reference)
import functools

import jax
import jax.numpy as jnp
from jax import lax
from jax.experimental import pallas as pl
from jax.experimental.pallas import tpu as pltpu
from jax.experimental.pallas import tpu_sc as plsc

_NC, _NS, _L = 2, 16, 16
_NW = _NC * _NS


def _knn3(xyz_n, xyz_s, nb):
    B, N, _ = xyz_n.shape
    S = xyz_s.shape[1]

    def body(xn_ref, xst_ref, i0, i1, i2, w0, w1, w2):
        b = pl.program_id(0)
        xn = xn_ref[0]
        xst = xst_ref[0]
        xn2 = (xn[:, 0:1] * xn[:, 0:1] + xn[:, 1:2] * xn[:, 1:2]) \
            + xn[:, 2:3] * xn[:, 2:3]
        xs2 = (xst[0:1] * xst[0:1] + xst[1:2] * xst[1:2]) \
            + xst[2:3] * xst[2:3]
        dot = lax.dot_general((-2.0 * xn).astype(jnp.bfloat16),
                              xst.astype(jnp.bfloat16),
                              (((1,), (0,)), ((), ())),
                              preferred_element_type=jnp.float32)
        d = (dot + xn2) + xs2
        lane = lax.broadcasted_iota(jnp.int32, d.shape, 1).astype(jnp.float32)
        idxs, dists = [], []
        for _ in range(3):
            m = jnp.min(d, axis=1, keepdims=True)
            ii = jnp.min(jnp.where(d == m, lane, jnp.float32(S)), axis=1,
                         keepdims=True)
            d = jnp.where(lane == ii, jnp.float32(jnp.inf), d)
            idxs.append(ii)
            dists.append(m)
        r = [1.0 / (m + 1e-8) for m in dists]
        norm = r[0] + r[1] + r[2]
        off = b * S
        for iref, wref, ii, rk in zip((i0, i1, i2), (w0, w1, w2), idxs, r):
            iref[0] = ii.astype(jnp.int32) + off
            wref[0] = rk / norm

    io = lambda b, n: (b, n, 0)
    so = lambda b, n: (b, 0, 0)
    xyz_s_t = xyz_s.transpose(0, 2, 1)
    outs = pl.pallas_call(
        body,
        grid=(B, N // nb),
        in_specs=[pl.BlockSpec((1, nb, 3), io), pl.BlockSpec((1, 3, S), so)],
        out_specs=[pl.BlockSpec((1, nb, 1), io)] * 6,
        out_shape=[jax.ShapeDtypeStruct((B, N, 1), jnp.int32)] * 3
        + [jax.ShapeDtypeStruct((B, N, 1), jnp.float32)] * 3,
    )(xyz_n, xyz_s_t)
    M = B * N
    return tuple(x.reshape(M) for x in outs)


def _interp(table, i0, i1, i2, w0, w1, w2, C=16):
    M = i0.shape[0]
    D = table.shape[1]
    P = M // _NW
    nch = P // C
    assert nch % 2 == 0
    mesh = plsc.VectorSubcoreMesh(core_axis_name="c", subcore_axis_name="s",
                                  num_cores=_NC, num_subcores=_NS)

    def body(tab, i0r, i1r, i2r, w0r, w1r, w2r, out,
             iv0, iv1, iv2, wv0, wv1, wv2, r0, r1, r2, ob, sems):
        wid = lax.axis_index("s") * _NC + lax.axis_index("c")
        base = wid * P
        ivs, wvs, rs = (iv0, iv1, iv2), (wv0, wv1, wv2), (r0, r1, r2)

        def fire(s, buf):
            off = base + s * C
            for k in range(3):
                pltpu.sync_copy((i0r, i1r, i2r)[k].at[pl.ds(off, C)],
                                ivs[k].at[buf])
                pltpu.sync_copy((w0r, w1r, w2r)[k].at[pl.ds(off, C)],
                                wvs[k].at[buf])
            for k in range(3):
                pltpu.async_copy(tab.at[ivs[k].at[buf]], rs[k].at[buf],
                                 sems.at[buf, k])

        def drain(buf):
            for k in range(3):
                pltpu.make_async_copy(tab.at[ivs[k].at[buf]], rs[k].at[buf],
                                      sems.at[buf, k]).wait()

        def compute(s, buf):
            cb = jnp.full((_L,), buf, jnp.int32)

            @plsc.parallel_loop(0, C, unroll=8)
            def _(c):
                cs = jnp.full((_L,), c, jnp.int32)
                a0 = plsc.load_gather(wv0, [cb, cs])
                a1 = plsc.load_gather(wv1, [cb, cs])
                a2 = plsc.load_gather(wv2, [cb, cs])
                for j in range(D // _L):
                    sl = pl.ds(j * _L, _L)
                    ob[c, sl] = (a0 * r0[buf, c, sl]
                                 + a1 * r1[buf, c, sl]
                                 + a2 * r2[buf, c, sl])

            pltpu.sync_copy(ob, out.at[pl.ds(base + s * C, C)])

        fire(0, 0)

        def pair(g, carry):
            s0 = 2 * g
            fire(s0 + 1, 1)
            drain(0)
            compute(s0, 0)

            @pl.when(g + 1 < nch // 2)
            def _():
                fire(s0 + 2, 0)

            drain(1)
            compute(s0 + 1, 1)
            return carry

        lax.fori_loop(0, nch // 2, pair, 0)

    kfn = pl.kernel(
        body,
        out_type=jax.ShapeDtypeStruct((M, D), jnp.float32),
        mesh=mesh,
        compiler_params=pltpu.CompilerParams(needs_layout_passes=False),
        scratch_types=[
            pltpu.VMEM((2, C), jnp.int32),
            pltpu.VMEM((2, C), jnp.int32),
            pltpu.VMEM((2, C), jnp.int32),
            pltpu.VMEM((2, C), jnp.float32),
            pltpu.VMEM((2, C), jnp.float32),
            pltpu.VMEM((2, C), jnp.float32),
            pltpu.VMEM((2, C, D), jnp.float32),
            pltpu.VMEM((2, C, D), jnp.float32),
            pltpu.VMEM((2, C, D), jnp.float32),
            pltpu.VMEM((C, D), jnp.float32),
            pltpu.SemaphoreType.DMA((2, 3)),
        ],
    )
    return kfn(table, i0, i1, i2, w0, w1, w2)


def _mm2_stats(xa, xb, wa, wb, rb=512):
    M, Ka = xa.shape
    Kb = xb.shape[1]
    Cout = wa.shape[0]

    def body(xa_ref, xb_ref, wa_ref, wb_ref, z_ref, s_ref, q_ref):
        z = lax.dot_general(xa_ref[...].astype(jnp.bfloat16),
                            wa_ref[...].astype(jnp.bfloat16),
                            (((1,), (1,)), ((), ())),
                            preferred_element_type=jnp.float32)
        z = z + lax.dot_general(xb_ref[...].astype(jnp.bfloat16),
                                wb_ref[...].astype(jnp.bfloat16),
                                (((1,), (1,)), ((), ())),
                                preferred_element_type=jnp.float32)
        z_ref[...] = z

        @pl.when(pl.program_id(0) == 0)
        def _():
            s_ref[...] = jnp.zeros_like(s_ref)
            q_ref[...] = jnp.zeros_like(q_ref)

        s_ref[...] += jnp.sum(z, axis=0, keepdims=True)
        q_ref[...] += jnp.sum(z * z, axis=0, keepdims=True)

    return pl.pallas_call(
        body,
        grid=(M // rb,),
        in_specs=[
            pl.BlockSpec((rb, Ka), lambda i: (i, 0)),
            pl.BlockSpec((rb, Kb), lambda i: (i, 0)),
            pl.BlockSpec((Cout, Ka), lambda i: (0, 0)),
            pl.BlockSpec((Cout, Kb), lambda i: (0, 0)),
        ],
        out_specs=[
            pl.BlockSpec((rb, Cout), lambda i: (i, 0)),
            pl.BlockSpec((1, Cout), lambda i: (0, 0)),
            pl.BlockSpec((1, Cout), lambda i: (0, 0)),
        ],
        out_shape=[
            jax.ShapeDtypeStruct((M, Cout), jnp.float32),
            jax.ShapeDtypeStruct((1, Cout), jnp.float32),
            jax.ShapeDtypeStruct((1, Cout), jnp.float32),
        ],
    )(xa, xb, wa, wb)


def _bn_scale_shift(s_ref, q_ref, g_ref, b_ref, m):
    mean = s_ref[...] * (1.0 / m)
    var = q_ref[...] * (1.0 / m) - mean * mean
    sc = g_ref[...] * lax.rsqrt(var + 1e-5)
    sh = b_ref[...] - mean * sc
    return sc, sh


def _bn_relu(z, s, q, g, b, rb=512):
    M, C = z.shape

    def body(z_ref, s_ref, q_ref, g_ref, b_ref, y_ref):
        sc, sh = _bn_scale_shift(s_ref, q_ref, g_ref, b_ref, M)
        y_ref[...] = jnp.maximum(z_ref[...] * sc + sh, 0.0)

    return pl.pallas_call(
        body,
        grid=(M // rb,),
        in_specs=[
            pl.BlockSpec((rb, C), lambda i: (i, 0)),
            pl.BlockSpec((1, C), lambda i: (0, 0)),
            pl.BlockSpec((1, C), lambda i: (0, 0)),
            pl.BlockSpec((1, C), lambda i: (0, 0)),
            pl.BlockSpec((1, C), lambda i: (0, 0)),
        ],
        out_specs=pl.BlockSpec((rb, C), lambda i: (i, 0)),
        out_shape=jax.ShapeDtypeStruct((M, C), jnp.float32),
    )(z, s, q, g, b)


def _bn_mm_stats(z, s, q, g, b, w, rb=512):
    M, C = z.shape
    Cout = w.shape[0]

    def body(z_ref, s_ref, q_ref, g_ref, b_ref, w_ref, z2_ref, s2_ref, q2_ref):
        sc, sh = _bn_scale_shift(s_ref, q_ref, g_ref, b_ref, M)
        h = jnp.maximum(z_ref[...] * sc + sh, 0.0)
        z2 = lax.dot_general(h.astype(jnp.bfloat16),
                             w_ref[...].astype(jnp.bfloat16),
                             (((1,), (1,)), ((), ())),
                             preferred_element_type=jnp.float32)
        z2_ref[...] = z2

        @pl.when(pl.program_id(0) == 0)
        def _():
            s2_ref[...] = jnp.zeros_like(s2_ref)
            q2_ref[...] = jnp.zeros_like(q2_ref)

        s2_ref[...] += jnp.sum(z2, axis=0, keepdims=True)
        q2_ref[...] += jnp.sum(z2 * z2, axis=0, keepdims=True)

    return pl.pallas_call(
        body,
        grid=(M // rb,),
        in_specs=[
            pl.BlockSpec((rb, C), lambda i: (i, 0)),
            pl.BlockSpec((1, C), lambda i: (0, 0)),
            pl.BlockSpec((1, C), lambda i: (0, 0)),
            pl.BlockSpec((1, C), lambda i: (0, 0)),
            pl.BlockSpec((1, C), lambda i: (0, 0)),
            pl.BlockSpec((Cout, C), lambda i: (0, 0)),
        ],
        out_specs=[
            pl.BlockSpec((rb, Cout), lambda i: (i, 0)),
            pl.BlockSpec((1, Cout), lambda i: (0, 0)),
            pl.BlockSpec((1, Cout), lambda i: (0, 0)),
        ],
        out_shape=[
            jax.ShapeDtypeStruct((M, Cout), jnp.float32),
            jax.ShapeDtypeStruct((1, Cout), jnp.float32),
            jax.ShapeDtypeStruct((1, Cout), jnp.float32),
        ],
    )(z, s, q, g, b, w)


def _bn_vec(z, s, q, g, b, w, rb=512):
    M, C = z.shape

    def body(z_ref, s_ref, q_ref, g_ref, b_ref, w_ref, o_ref):
        sc, sh = _bn_scale_shift(s_ref, q_ref, g_ref, b_ref, M)
        h = jnp.maximum(z_ref[...] * sc + sh, 0.0)
        o_ref[...] = jnp.sum(h * w_ref[...], axis=1, keepdims=True)

    return pl.pallas_call(
        body,
        grid=(M // rb,),
        in_specs=[
            pl.BlockSpec((rb, C), lambda i: (i, 0)),
            pl.BlockSpec((1, C), lambda i: (0, 0)),
            pl.BlockSpec((1, C), lambda i: (0, 0)),
            pl.BlockSpec((1, C), lambda i: (0, 0)),
            pl.BlockSpec((1, C), lambda i: (0, 0)),
            pl.BlockSpec((1, C), lambda i: (0, 0)),
        ],
        out_specs=pl.BlockSpec((rb, 1), lambda i: (i, 0)),
        out_shape=jax.ShapeDtypeStruct((M, 1), jnp.float32),
    )(z, s, q, g, b, w)


def kernel(xyz_a, xyz_b, xyz_c, feat_a, feat_b, feat_c,
           W_lin0, g0, b0, W_lin1, g1, b1, W_dec0, gd, bd, W_dec1):
    B, Na, _ = xyz_a.shape
    Nb_ = xyz_b.shape[1]
    Sc = xyz_c.shape[1]
    D1, D2, D3 = feat_a.shape[1], feat_b.shape[1], feat_c.shape[1]

    faT = feat_a.transpose(0, 2, 1).reshape(B * Na, D1)
    fbT = feat_b.transpose(0, 2, 1).reshape(B * Nb_, D2)
    fcT = feat_c.transpose(0, 2, 1).reshape(B * Sc, D3)

    i0, i1, i2, w0, w1, w2 = _knn3(xyz_b, xyz_c, 256)
    j0, j1, j2, v0, v1, v2 = _knn3(xyz_a, xyz_b, 256)

    interp0 = _interp(fcT, i0, i1, i2, w0, w1, w2)
    z0, s0_, q0_ = _mm2_stats(fbT, interp0, W_lin0[:, :D2], W_lin0[:, D2:])
    y0 = _bn_relu(z0, s0_, q0_, g0.reshape(1, -1), b0.reshape(1, -1))

    interp1 = _interp(y0, j0, j1, j2, v0, v1, v2)
    z1, s1_, q1_ = _mm2_stats(faT, interp1, W_lin1[:, :D1], W_lin1[:, D1:])
    z2, s2_, q2_ = _bn_mm_stats(z1, s1_, q1_, g1.reshape(1, -1),
                                b1.reshape(1, -1), W_dec0)
    out = _bn_vec(z2, s2_, q2_, gd.reshape(1, -1), bd.reshape(1, -1), W_dec1)
    return out.reshape(B, Na)


# speedup vs baseline: 1.2084x; 1.2084x over previous
import functools

import jax
import jax.numpy as jnp
from jax import lax
from jax.experimental import pallas as pl
from jax.experimental.pallas import tpu as pltpu
from jax.experimental.pallas import tpu_sc as plsc

_NC, _NS, _L = 2, 16, 16
_NW = _NC * _NS


def _knn3(xyz_n, xyz_s, nb):
    B, N, _ = xyz_n.shape
    S = xyz_s.shape[1]

    def body(xn_ref, xst_ref, i0, i1, i2, w0, w1, w2):
        b = pl.program_id(0)
        xn = xn_ref[0]
        xst = xst_ref[0]
        xn2 = (xn[:, 0:1] * xn[:, 0:1] + xn[:, 1:2] * xn[:, 1:2]) \
            + xn[:, 2:3] * xn[:, 2:3]
        xs2 = (xst[0:1] * xst[0:1] + xst[1:2] * xst[1:2]) \
            + xst[2:3] * xst[2:3]
        dot = lax.dot_general((-2.0 * xn).astype(jnp.bfloat16),
                              xst.astype(jnp.bfloat16),
                              (((1,), (0,)), ((), ())),
                              preferred_element_type=jnp.float32)
        d = (dot + xn2) + xs2
        lane = lax.broadcasted_iota(jnp.int32, d.shape, 1).astype(jnp.float32)
        idxs, dists = [], []
        for _ in range(3):
            m = jnp.min(d, axis=1, keepdims=True)
            ii = jnp.min(jnp.where(d == m, lane, jnp.float32(S)), axis=1,
                         keepdims=True)
            d = jnp.where(lane == ii, jnp.float32(jnp.inf), d)
            idxs.append(ii)
            dists.append(m)
        r = [1.0 / (m + 1e-8) for m in dists]
        norm = r[0] + r[1] + r[2]
        off = b * S
        for iref, wref, ii, rk in zip((i0, i1, i2), (w0, w1, w2), idxs, r):
            iref[0] = ii.astype(jnp.int32) + off
            wref[0] = rk / norm

    io = lambda b, n: (b, n, 0)
    so = lambda b, n: (b, 0, 0)
    xyz_s_t = xyz_s.transpose(0, 2, 1)
    outs = pl.pallas_call(
        body,
        grid=(B, N // nb),
        in_specs=[pl.BlockSpec((1, nb, 3), io), pl.BlockSpec((1, 3, S), so)],
        out_specs=[pl.BlockSpec((1, nb, 1), io)] * 6,
        out_shape=[jax.ShapeDtypeStruct((B, N, 1), jnp.int32)] * 3
        + [jax.ShapeDtypeStruct((B, N, 1), jnp.float32)] * 3,
    )(xyz_n, xyz_s_t)
    M = B * N
    return tuple(x.reshape(M) for x in outs)


def _interp(table, i0, i1, i2, w0, w1, w2, C=16):
    M = i0.shape[0]
    D = table.shape[1]
    P = M // _NW
    nch = P // C
    assert nch % 2 == 0
    mesh = plsc.VectorSubcoreMesh(core_axis_name="c", subcore_axis_name="s",
                                  num_cores=_NC, num_subcores=_NS)

    def body(tab, i0r, i1r, i2r, w0r, w1r, w2r, out,
             iv0, iv1, iv2, wv0, wv1, wv2, r0, r1, r2, ob, sems):
        wid = lax.axis_index("s") * _NC + lax.axis_index("c")
        base = wid * P
        ivs, wvs, rs = (iv0, iv1, iv2), (wv0, wv1, wv2), (r0, r1, r2)

        def fire(s, buf):
            off = base + s * C
            for k in range(3):
                pltpu.sync_copy((i0r, i1r, i2r)[k].at[pl.ds(off, C)],
                                ivs[k].at[buf])
                pltpu.sync_copy((w0r, w1r, w2r)[k].at[pl.ds(off, C)],
                                wvs[k].at[buf])
            for k in range(3):
                pltpu.async_copy(tab.at[ivs[k].at[buf]], rs[k].at[buf],
                                 sems.at[buf, k])

        def drain(buf):
            for k in range(3):
                pltpu.make_async_copy(tab.at[ivs[k].at[buf]], rs[k].at[buf],
                                      sems.at[buf, k]).wait()

        def compute(s, buf):
            cb = jnp.full((_L,), buf, jnp.int32)

            @plsc.parallel_loop(0, C, unroll=4)
            def _(c):
                cs = jnp.full((_L,), c, jnp.int32)
                a0 = plsc.load_gather(wv0, [cb, cs])
                a1 = plsc.load_gather(wv1, [cb, cs])
                a2 = plsc.load_gather(wv2, [cb, cs])
                for j in range(D // _L):
                    sl = pl.ds(j * _L, _L)
                    ob[c, sl] = (a0 * r0[buf, c, sl]
                                 + a1 * r1[buf, c, sl]
                                 + a2 * r2[buf, c, sl])

            pltpu.sync_copy(ob, out.at[pl.ds(base + s * C, C)])

        fire(0, 0)

        def pair(g, carry):
            s0 = 2 * g
            fire(s0 + 1, 1)
            drain(0)
            compute(s0, 0)

            @pl.when(g + 1 < nch // 2)
            def _():
                fire(s0 + 2, 0)

            drain(1)
            compute(s0 + 1, 1)
            return carry

        lax.fori_loop(0, nch // 2, pair, 0)

    kfn = pl.kernel(
        body,
        out_type=jax.ShapeDtypeStruct((M, D), jnp.float32),
        mesh=mesh,
        compiler_params=pltpu.CompilerParams(needs_layout_passes=False),
        scratch_types=[
            pltpu.VMEM((2, C), jnp.int32),
            pltpu.VMEM((2, C), jnp.int32),
            pltpu.VMEM((2, C), jnp.int32),
            pltpu.VMEM((2, C), jnp.float32),
            pltpu.VMEM((2, C), jnp.float32),
            pltpu.VMEM((2, C), jnp.float32),
            pltpu.VMEM((2, C, D), jnp.float32),
            pltpu.VMEM((2, C, D), jnp.float32),
            pltpu.VMEM((2, C, D), jnp.float32),
            pltpu.VMEM((C, D), jnp.float32),
            pltpu.SemaphoreType.DMA((2, 3)),
        ],
    )
    return kfn(table, i0, i1, i2, w0, w1, w2)


def _mm2_stats(xa, xb, wa, wb, rb=512):
    M, Ka = xa.shape
    Kb = xb.shape[1]
    Cout = wa.shape[0]

    def body(xa_ref, xb_ref, wa_ref, wb_ref, z_ref, s_ref, q_ref):
        z = lax.dot_general(xa_ref[...].astype(jnp.bfloat16),
                            wa_ref[...].astype(jnp.bfloat16),
                            (((1,), (1,)), ((), ())),
                            preferred_element_type=jnp.float32)
        z = z + lax.dot_general(xb_ref[...].astype(jnp.bfloat16),
                                wb_ref[...].astype(jnp.bfloat16),
                                (((1,), (1,)), ((), ())),
                                preferred_element_type=jnp.float32)
        z_ref[...] = z

        @pl.when(pl.program_id(0) == 0)
        def _():
            s_ref[...] = jnp.zeros_like(s_ref)
            q_ref[...] = jnp.zeros_like(q_ref)

        s_ref[...] += jnp.sum(z, axis=0, keepdims=True)
        q_ref[...] += jnp.sum(z * z, axis=0, keepdims=True)

    return pl.pallas_call(
        body,
        grid=(M // rb,),
        in_specs=[
            pl.BlockSpec((rb, Ka), lambda i: (i, 0)),
            pl.BlockSpec((rb, Kb), lambda i: (i, 0)),
            pl.BlockSpec((Cout, Ka), lambda i: (0, 0)),
            pl.BlockSpec((Cout, Kb), lambda i: (0, 0)),
        ],
        out_specs=[
            pl.BlockSpec((rb, Cout), lambda i: (i, 0)),
            pl.BlockSpec((1, Cout), lambda i: (0, 0)),
            pl.BlockSpec((1, Cout), lambda i: (0, 0)),
        ],
        out_shape=[
            jax.ShapeDtypeStruct((M, Cout), jnp.float32),
            jax.ShapeDtypeStruct((1, Cout), jnp.float32),
            jax.ShapeDtypeStruct((1, Cout), jnp.float32),
        ],
    )(xa, xb, wa, wb)


def _bn_scale_shift(s_ref, q_ref, g_ref, b_ref, m):
    mean = s_ref[...] * (1.0 / m)
    var = q_ref[...] * (1.0 / m) - mean * mean
    sc = g_ref[...] * lax.rsqrt(var + 1e-5)
    sh = b_ref[...] - mean * sc
    return sc, sh


def _bn_relu(z, s, q, g, b, rb=512):
    M, C = z.shape

    def body(z_ref, s_ref, q_ref, g_ref, b_ref, y_ref):
        sc, sh = _bn_scale_shift(s_ref, q_ref, g_ref, b_ref, M)
        y_ref[...] = jnp.maximum(z_ref[...] * sc + sh, 0.0)

    return pl.pallas_call(
        body,
        grid=(M // rb,),
        in_specs=[
            pl.BlockSpec((rb, C), lambda i: (i, 0)),
            pl.BlockSpec((1, C), lambda i: (0, 0)),
            pl.BlockSpec((1, C), lambda i: (0, 0)),
            pl.BlockSpec((1, C), lambda i: (0, 0)),
            pl.BlockSpec((1, C), lambda i: (0, 0)),
        ],
        out_specs=pl.BlockSpec((rb, C), lambda i: (i, 0)),
        out_shape=jax.ShapeDtypeStruct((M, C), jnp.float32),
    )(z, s, q, g, b)


def _bn_mm_stats(z, s, q, g, b, w, rb=512):
    M, C = z.shape
    Cout = w.shape[0]

    def body(z_ref, s_ref, q_ref, g_ref, b_ref, w_ref, z2_ref, s2_ref, q2_ref):
        sc, sh = _bn_scale_shift(s_ref, q_ref, g_ref, b_ref, M)
        h = jnp.maximum(z_ref[...] * sc + sh, 0.0)
        z2 = lax.dot_general(h.astype(jnp.bfloat16),
                             w_ref[...].astype(jnp.bfloat16),
                             (((1,), (1,)), ((), ())),
                             preferred_element_type=jnp.float32)
        z2_ref[...] = z2

        @pl.when(pl.program_id(0) == 0)
        def _():
            s2_ref[...] = jnp.zeros_like(s2_ref)
            q2_ref[...] = jnp.zeros_like(q2_ref)

        s2_ref[...] += jnp.sum(z2, axis=0, keepdims=True)
        q2_ref[...] += jnp.sum(z2 * z2, axis=0, keepdims=True)

    return pl.pallas_call(
        body,
        grid=(M // rb,),
        in_specs=[
            pl.BlockSpec((rb, C), lambda i: (i, 0)),
            pl.BlockSpec((1, C), lambda i: (0, 0)),
            pl.BlockSpec((1, C), lambda i: (0, 0)),
            pl.BlockSpec((1, C), lambda i: (0, 0)),
            pl.BlockSpec((1, C), lambda i: (0, 0)),
            pl.BlockSpec((Cout, C), lambda i: (0, 0)),
        ],
        out_specs=[
            pl.BlockSpec((rb, Cout), lambda i: (i, 0)),
            pl.BlockSpec((1, Cout), lambda i: (0, 0)),
            pl.BlockSpec((1, Cout), lambda i: (0, 0)),
        ],
        out_shape=[
            jax.ShapeDtypeStruct((M, Cout), jnp.float32),
            jax.ShapeDtypeStruct((1, Cout), jnp.float32),
            jax.ShapeDtypeStruct((1, Cout), jnp.float32),
        ],
    )(z, s, q, g, b, w)


def _bn_vec(z, s, q, g, b, w, rb=512):
    M, C = z.shape

    def body(z_ref, s_ref, q_ref, g_ref, b_ref, w_ref, o_ref):
        sc, sh = _bn_scale_shift(s_ref, q_ref, g_ref, b_ref, M)
        h = jnp.maximum(z_ref[...] * sc + sh, 0.0)
        o_ref[...] = jnp.sum(h * w_ref[...], axis=1, keepdims=True)

    return pl.pallas_call(
        body,
        grid=(M // rb,),
        in_specs=[
            pl.BlockSpec((rb, C), lambda i: (i, 0)),
            pl.BlockSpec((1, C), lambda i: (0, 0)),
            pl.BlockSpec((1, C), lambda i: (0, 0)),
            pl.BlockSpec((1, C), lambda i: (0, 0)),
            pl.BlockSpec((1, C), lambda i: (0, 0)),
            pl.BlockSpec((1, C), lambda i: (0, 0)),
        ],
        out_specs=pl.BlockSpec((rb, 1), lambda i: (i, 0)),
        out_shape=jax.ShapeDtypeStruct((M, 1), jnp.float32),
    )(z, s, q, g, b, w)


def kernel(xyz_a, xyz_b, xyz_c, feat_a, feat_b, feat_c,
           W_lin0, g0, b0, W_lin1, g1, b1, W_dec0, gd, bd, W_dec1):
    B, Na, _ = xyz_a.shape
    Nb_ = xyz_b.shape[1]
    Sc = xyz_c.shape[1]
    D1, D2, D3 = feat_a.shape[1], feat_b.shape[1], feat_c.shape[1]

    faT = feat_a.transpose(0, 2, 1).reshape(B * Na, D1)
    fbT = feat_b.transpose(0, 2, 1).reshape(B * Nb_, D2)
    fcT = feat_c.transpose(0, 2, 1).reshape(B * Sc, D3)

    i0, i1, i2, w0, w1, w2 = _knn3(xyz_b, xyz_c, 256)
    j0, j1, j2, v0, v1, v2 = _knn3(xyz_a, xyz_b, 256)

    interp0 = _interp(fcT, i0, i1, i2, w0, w1, w2)
    z0, s0_, q0_ = _mm2_stats(fbT, interp0, W_lin0[:, :D2], W_lin0[:, D2:])
    y0 = _bn_relu(z0, s0_, q0_, g0.reshape(1, -1), b0.reshape(1, -1))

    interp1 = _interp(y0, j0, j1, j2, v0, v1, v2)
    z1, s1_, q1_ = _mm2_stats(faT, interp1, W_lin1[:, :D1], W_lin1[:, D1:])
    z2, s2_, q2_ = _bn_mm_stats(z1, s1_, q1_, g1.reshape(1, -1),
                                b1.reshape(1, -1), W_dec0)
    out = _bn_vec(z2, s2_, q2_, gd.reshape(1, -1), bd.reshape(1, -1), W_dec1)
    return out.reshape(B, Na)


# speedup vs baseline: 1.2389x; 1.0253x over previous
import functools

import jax
import jax.numpy as jnp
from jax import lax
from jax.experimental import pallas as pl
from jax.experimental.pallas import tpu as pltpu
from jax.experimental.pallas import tpu_sc as plsc

_NC, _NS, _L = 2, 16, 16
_NW = _NC * _NS


def _knn3(xyz_n, xyz_s, nb):
    B, N, _ = xyz_n.shape
    S = xyz_s.shape[1]

    def body(xn_ref, xst_ref, i0, i1, i2, w0, w1, w2):
        b = pl.program_id(0)
        xn = xn_ref[0]
        xst = xst_ref[0]
        xn2 = (xn[:, 0:1] * xn[:, 0:1] + xn[:, 1:2] * xn[:, 1:2]) \
            + xn[:, 2:3] * xn[:, 2:3]
        xs2 = (xst[0:1] * xst[0:1] + xst[1:2] * xst[1:2]) \
            + xst[2:3] * xst[2:3]
        dot = lax.dot_general((-2.0 * xn).astype(jnp.bfloat16),
                              xst.astype(jnp.bfloat16),
                              (((1,), (0,)), ((), ())),
                              preferred_element_type=jnp.float32)
        d = (dot + xn2) + xs2
        lane = lax.broadcasted_iota(jnp.int32, d.shape, 1).astype(jnp.float32)
        idxs, dists = [], []
        for _ in range(3):
            m = jnp.min(d, axis=1, keepdims=True)
            ii = jnp.min(jnp.where(d == m, lane, jnp.float32(S)), axis=1,
                         keepdims=True)
            d = jnp.where(lane == ii, jnp.float32(jnp.inf), d)
            idxs.append(ii)
            dists.append(m)
        r = [1.0 / (m + 1e-8) for m in dists]
        norm = r[0] + r[1] + r[2]
        off = b * S
        for iref, wref, ii, rk in zip((i0, i1, i2), (w0, w1, w2), idxs, r):
            iref[0] = ii.astype(jnp.int32) + off
            wref[0] = rk / norm

    io = lambda b, n: (b, n, 0)
    so = lambda b, n: (b, 0, 0)
    xyz_s_t = xyz_s.transpose(0, 2, 1)
    outs = pl.pallas_call(
        body,
        grid=(B, N // nb),
        in_specs=[pl.BlockSpec((1, nb, 3), io), pl.BlockSpec((1, 3, S), so)],
        out_specs=[pl.BlockSpec((1, nb, 1), io)] * 6,
        out_shape=[jax.ShapeDtypeStruct((B, N, 1), jnp.int32)] * 3
        + [jax.ShapeDtypeStruct((B, N, 1), jnp.float32)] * 3,
    )(xyz_n, xyz_s_t)
    M = B * N
    return tuple(x.reshape(M) for x in outs)


def _interp(table, i0, i1, i2, w0, w1, w2, C=16):
    M = i0.shape[0]
    D = table.shape[1]
    P = M // _NW
    nch = P // C
    assert nch % 2 == 0
    mesh = plsc.VectorSubcoreMesh(core_axis_name="c", subcore_axis_name="s",
                                  num_cores=_NC, num_subcores=_NS)

    def body(tab, i0r, i1r, i2r, w0r, w1r, w2r, out,
             iv0, iv1, iv2, wv0, wv1, wv2, r0, r1, r2, ob, sems):
        wid = lax.axis_index("s") * _NC + lax.axis_index("c")
        base = wid * P
        ivs, wvs, rs = (iv0, iv1, iv2), (wv0, wv1, wv2), (r0, r1, r2)

        def fire(s, buf):
            off = base + s * C
            for k in range(3):
                pltpu.sync_copy((i0r, i1r, i2r)[k].at[pl.ds(off, C)],
                                ivs[k].at[buf])
                pltpu.sync_copy((w0r, w1r, w2r)[k].at[pl.ds(off, C)],
                                wvs[k].at[buf])
            for k in range(3):
                pltpu.async_copy(tab.at[ivs[k].at[buf]], rs[k].at[buf],
                                 sems.at[buf, k])

        def drain(buf):
            for k in range(3):
                pltpu.make_async_copy(tab.at[ivs[k].at[buf]], rs[k].at[buf],
                                      sems.at[buf, k]).wait()

        def compute(s, buf):
            cb = jnp.full((_L,), buf, jnp.int32)

            @plsc.parallel_loop(0, C, unroll=4)
            def _(c):
                cs = jnp.full((_L,), c, jnp.int32)
                a0 = plsc.load_gather(wv0, [cb, cs])
                a1 = plsc.load_gather(wv1, [cb, cs])
                a2 = plsc.load_gather(wv2, [cb, cs])
                for j in range(D // _L):
                    sl = pl.ds(j * _L, _L)
                    ob[c, sl] = (a0 * r0[buf, c, sl]
                                 + a1 * r1[buf, c, sl]
                                 + a2 * r2[buf, c, sl])

            pltpu.sync_copy(ob, out.at[pl.ds(base + s * C, C)])

        fire(0, 0)

        def pair(g, carry):
            s0 = 2 * g
            fire(s0 + 1, 1)
            drain(0)
            compute(s0, 0)

            @pl.when(g + 1 < nch // 2)
            def _():
                fire(s0 + 2, 0)

            drain(1)
            compute(s0 + 1, 1)
            return carry

        lax.fori_loop(0, nch // 2, pair, 0)

    kfn = pl.kernel(
        body,
        out_type=jax.ShapeDtypeStruct((M, D), jnp.float32),
        mesh=mesh,
        compiler_params=pltpu.CompilerParams(needs_layout_passes=False),
        scratch_types=[
            pltpu.VMEM((2, C), jnp.int32),
            pltpu.VMEM((2, C), jnp.int32),
            pltpu.VMEM((2, C), jnp.int32),
            pltpu.VMEM((2, C), jnp.float32),
            pltpu.VMEM((2, C), jnp.float32),
            pltpu.VMEM((2, C), jnp.float32),
            pltpu.VMEM((2, C, D), jnp.float32),
            pltpu.VMEM((2, C, D), jnp.float32),
            pltpu.VMEM((2, C, D), jnp.float32),
            pltpu.VMEM((C, D), jnp.float32),
            pltpu.SemaphoreType.DMA((2, 3)),
        ],
    )
    return kfn(table, i0, i1, i2, w0, w1, w2)


def _mm2_stats(xa, xb, wa, wb, rb=512):
    M, Ka = xa.shape
    Kb = xb.shape[1]
    Cout = wa.shape[0]

    def body(xa_ref, xb_ref, wa_ref, wb_ref, z_ref, s_ref, q_ref):
        z = lax.dot_general(xa_ref[...].astype(jnp.bfloat16),
                            wa_ref[...].astype(jnp.bfloat16),
                            (((1,), (1,)), ((), ())),
                            preferred_element_type=jnp.float32)
        z = z + lax.dot_general(xb_ref[...].astype(jnp.bfloat16),
                                wb_ref[...].astype(jnp.bfloat16),
                                (((1,), (1,)), ((), ())),
                                preferred_element_type=jnp.float32)
        z_ref[...] = z

        @pl.when(pl.program_id(0) == 0)
        def _():
            s_ref[...] = jnp.zeros_like(s_ref)
            q_ref[...] = jnp.zeros_like(q_ref)

        s_ref[...] += jnp.sum(z, axis=0, keepdims=True)
        q_ref[...] += jnp.sum(z * z, axis=0, keepdims=True)

    return pl.pallas_call(
        body,
        grid=(M // rb,),
        in_specs=[
            pl.BlockSpec((rb, Ka), lambda i: (i, 0)),
            pl.BlockSpec((rb, Kb), lambda i: (i, 0)),
            pl.BlockSpec((Cout, Ka), lambda i: (0, 0)),
            pl.BlockSpec((Cout, Kb), lambda i: (0, 0)),
        ],
        out_specs=[
            pl.BlockSpec((rb, Cout), lambda i: (i, 0)),
            pl.BlockSpec((1, Cout), lambda i: (0, 0)),
            pl.BlockSpec((1, Cout), lambda i: (0, 0)),
        ],
        out_shape=[
            jax.ShapeDtypeStruct((M, Cout), jnp.float32),
            jax.ShapeDtypeStruct((1, Cout), jnp.float32),
            jax.ShapeDtypeStruct((1, Cout), jnp.float32),
        ],
    )(xa, xb, wa, wb)


def _bn_scale_shift(s_ref, q_ref, g_ref, b_ref, m):
    mean = s_ref[...] * (1.0 / m)
    var = q_ref[...] * (1.0 / m) - mean * mean
    sc = g_ref[...] * lax.rsqrt(var + 1e-5)
    sh = b_ref[...] - mean * sc
    return sc, sh


def _bn_relu(z, s, q, g, b, rb=512):
    M, C = z.shape

    def body(z_ref, s_ref, q_ref, g_ref, b_ref, y_ref):
        sc, sh = _bn_scale_shift(s_ref, q_ref, g_ref, b_ref, M)
        y_ref[...] = jnp.maximum(z_ref[...] * sc + sh, 0.0)

    return pl.pallas_call(
        body,
        grid=(M // rb,),
        in_specs=[
            pl.BlockSpec((rb, C), lambda i: (i, 0)),
            pl.BlockSpec((1, C), lambda i: (0, 0)),
            pl.BlockSpec((1, C), lambda i: (0, 0)),
            pl.BlockSpec((1, C), lambda i: (0, 0)),
            pl.BlockSpec((1, C), lambda i: (0, 0)),
        ],
        out_specs=pl.BlockSpec((rb, C), lambda i: (i, 0)),
        out_shape=jax.ShapeDtypeStruct((M, C), jnp.float32),
    )(z, s, q, g, b)


def _bn_mm_stats(z, s, q, g, b, w, rb=512):
    M, C = z.shape
    Cout = w.shape[0]

    def body(z_ref, s_ref, q_ref, g_ref, b_ref, w_ref, z2_ref, s2_ref, q2_ref):
        sc, sh = _bn_scale_shift(s_ref, q_ref, g_ref, b_ref, M)
        h = jnp.maximum(z_ref[...] * sc + sh, 0.0)
        z2 = lax.dot_general(h.astype(jnp.bfloat16),
                             w_ref[...].astype(jnp.bfloat16),
                             (((1,), (1,)), ((), ())),
                             preferred_element_type=jnp.float32)
        z2_ref[...] = z2

        @pl.when(pl.program_id(0) == 0)
        def _():
            s2_ref[...] = jnp.zeros_like(s2_ref)
            q2_ref[...] = jnp.zeros_like(q2_ref)

        s2_ref[...] += jnp.sum(z2, axis=0, keepdims=True)
        q2_ref[...] += jnp.sum(z2 * z2, axis=0, keepdims=True)

    return pl.pallas_call(
        body,
        grid=(M // rb,),
        in_specs=[
            pl.BlockSpec((rb, C), lambda i: (i, 0)),
            pl.BlockSpec((1, C), lambda i: (0, 0)),
            pl.BlockSpec((1, C), lambda i: (0, 0)),
            pl.BlockSpec((1, C), lambda i: (0, 0)),
            pl.BlockSpec((1, C), lambda i: (0, 0)),
            pl.BlockSpec((Cout, C), lambda i: (0, 0)),
        ],
        out_specs=[
            pl.BlockSpec((rb, Cout), lambda i: (i, 0)),
            pl.BlockSpec((1, Cout), lambda i: (0, 0)),
            pl.BlockSpec((1, Cout), lambda i: (0, 0)),
        ],
        out_shape=[
            jax.ShapeDtypeStruct((M, Cout), jnp.float32),
            jax.ShapeDtypeStruct((1, Cout), jnp.float32),
            jax.ShapeDtypeStruct((1, Cout), jnp.float32),
        ],
    )(z, s, q, g, b, w)


def _bn_vec(z, s, q, g, b, w, rb=512):
    M, C = z.shape

    def body(z_ref, s_ref, q_ref, g_ref, b_ref, w_ref, o_ref):
        sc, sh = _bn_scale_shift(s_ref, q_ref, g_ref, b_ref, M)
        h = jnp.maximum(z_ref[...] * sc + sh, 0.0)
        o_ref[...] = jnp.sum(h * w_ref[...], axis=1, keepdims=True)

    return pl.pallas_call(
        body,
        grid=(M // rb,),
        in_specs=[
            pl.BlockSpec((rb, C), lambda i: (i, 0)),
            pl.BlockSpec((1, C), lambda i: (0, 0)),
            pl.BlockSpec((1, C), lambda i: (0, 0)),
            pl.BlockSpec((1, C), lambda i: (0, 0)),
            pl.BlockSpec((1, C), lambda i: (0, 0)),
            pl.BlockSpec((1, C), lambda i: (0, 0)),
        ],
        out_specs=pl.BlockSpec((rb, 1), lambda i: (i, 0)),
        out_shape=jax.ShapeDtypeStruct((M, 1), jnp.float32),
    )(z, s, q, g, b, w)


def kernel(xyz_a, xyz_b, xyz_c, feat_a, feat_b, feat_c,
           W_lin0, g0, b0, W_lin1, g1, b1, W_dec0, gd, bd, W_dec1):
    B, Na, _ = xyz_a.shape
    Nb_ = xyz_b.shape[1]
    Sc = xyz_c.shape[1]
    D1, D2, D3 = feat_a.shape[1], feat_b.shape[1], feat_c.shape[1]

    faT = feat_a.transpose(0, 2, 1).reshape(B * Na, D1)
    fbT = feat_b.transpose(0, 2, 1).reshape(B * Nb_, D2)
    fcT = feat_c.transpose(0, 2, 1).reshape(B * Sc, D3)

    i0, i1, i2, w0, w1, w2 = _knn3(xyz_b, xyz_c, 512)
    j0, j1, j2, v0, v1, v2 = _knn3(xyz_a, xyz_b, 512)

    interp0 = _interp(fcT, i0, i1, i2, w0, w1, w2)
    z0, s0_, q0_ = _mm2_stats(fbT, interp0, W_lin0[:, :D2], W_lin0[:, D2:])
    y0 = _bn_relu(z0, s0_, q0_, g0.reshape(1, -1), b0.reshape(1, -1))

    interp1 = _interp(y0, j0, j1, j2, v0, v1, v2)
    z1, s1_, q1_ = _mm2_stats(faT, interp1, W_lin1[:, :D1], W_lin1[:, D1:])
    z2, s2_, q2_ = _bn_mm_stats(z1, s1_, q1_, g1.reshape(1, -1),
                                b1.reshape(1, -1), W_dec0)
    out = _bn_vec(z2, s2_, q2_, gd.reshape(1, -1), bd.reshape(1, -1), W_dec1)
    return out.reshape(B, Na)


# speedup vs baseline: 1.2441x; 1.0042x over previous
import functools

import jax
import jax.numpy as jnp
from jax import lax
from jax.experimental import pallas as pl
from jax.experimental.pallas import tpu as pltpu
from jax.experimental.pallas import tpu_sc as plsc

_NC, _NS, _L = 2, 16, 16
_NW = _NC * _NS


def _knn3(xyz_n, xyz_s, nb):
    B, N, _ = xyz_n.shape
    S = xyz_s.shape[1]

    def body(xn_ref, xst_ref, i0, i1, i2, w0, w1, w2):
        b = pl.program_id(0)
        xn = xn_ref[0]
        xst = xst_ref[0]
        xn2 = (xn[:, 0:1] * xn[:, 0:1] + xn[:, 1:2] * xn[:, 1:2]) \
            + xn[:, 2:3] * xn[:, 2:3]
        xs2 = (xst[0:1] * xst[0:1] + xst[1:2] * xst[1:2]) \
            + xst[2:3] * xst[2:3]
        dot = lax.dot_general((-2.0 * xn).astype(jnp.bfloat16),
                              xst.astype(jnp.bfloat16),
                              (((1,), (0,)), ((), ())),
                              preferred_element_type=jnp.float32)
        d = (dot + xn2) + xs2
        lane = lax.broadcasted_iota(jnp.int32, d.shape, 1).astype(jnp.float32)
        idxs, dists = [], []
        for _ in range(3):
            m = jnp.min(d, axis=1, keepdims=True)
            ii = jnp.min(jnp.where(d == m, lane, jnp.float32(S)), axis=1,
                         keepdims=True)
            d = jnp.where(lane == ii, jnp.float32(jnp.inf), d)
            idxs.append(ii)
            dists.append(m)
        r = [1.0 / (m + 1e-8) for m in dists]
        norm = r[0] + r[1] + r[2]
        off = b * S
        for iref, wref, ii, rk in zip((i0, i1, i2), (w0, w1, w2), idxs, r):
            iref[0] = ii.astype(jnp.int32) + off
            wref[0] = rk / norm

    io = lambda b, n: (b, n, 0)
    so = lambda b, n: (b, 0, 0)
    xyz_s_t = xyz_s.transpose(0, 2, 1)
    outs = pl.pallas_call(
        body,
        grid=(B, N // nb),
        in_specs=[pl.BlockSpec((1, nb, 3), io), pl.BlockSpec((1, 3, S), so)],
        out_specs=[pl.BlockSpec((1, nb, 1), io)] * 6,
        out_shape=[jax.ShapeDtypeStruct((B, N, 1), jnp.int32)] * 3
        + [jax.ShapeDtypeStruct((B, N, 1), jnp.float32)] * 3,
    )(xyz_n, xyz_s_t)
    M = B * N
    return tuple(x.reshape(M) for x in outs)


def _interp(table, i0, i1, i2, w0, w1, w2, C=16):
    M = i0.shape[0]
    D = table.shape[1]
    P = M // _NW
    nch = P // C
    assert nch % 2 == 0
    mesh = plsc.VectorSubcoreMesh(core_axis_name="c", subcore_axis_name="s",
                                  num_cores=_NC, num_subcores=_NS)

    def body(tab, i0r, i1r, i2r, w0r, w1r, w2r, out,
             iv0, iv1, iv2, wv0, wv1, wv2, r0, r1, r2, ob, sems):
        wid = lax.axis_index("s") * _NC + lax.axis_index("c")
        base = wid * P
        ivs, wvs, rs = (iv0, iv1, iv2), (wv0, wv1, wv2), (r0, r1, r2)

        def fire(s, buf):
            off = base + s * C
            for k in range(3):
                pltpu.sync_copy((i0r, i1r, i2r)[k].at[pl.ds(off, C)],
                                ivs[k].at[buf])
                pltpu.sync_copy((w0r, w1r, w2r)[k].at[pl.ds(off, C)],
                                wvs[k].at[buf])
            for k in range(3):
                pltpu.async_copy(tab.at[ivs[k].at[buf]], rs[k].at[buf],
                                 sems.at[buf, k])

        def drain(buf):
            for k in range(3):
                pltpu.make_async_copy(tab.at[ivs[k].at[buf]], rs[k].at[buf],
                                      sems.at[buf, k]).wait()

        def compute(s, buf):
            cb = jnp.full((_L,), buf, jnp.int32)

            @plsc.parallel_loop(0, C, unroll=4)
            def _(c):
                cs = jnp.full((_L,), c, jnp.int32)
                a0 = plsc.load_gather(wv0, [cb, cs])
                a1 = plsc.load_gather(wv1, [cb, cs])
                a2 = plsc.load_gather(wv2, [cb, cs])
                for j in range(D // _L):
                    sl = pl.ds(j * _L, _L)
                    ob[c, sl] = (a0 * r0[buf, c, sl]
                                 + a1 * r1[buf, c, sl]
                                 + a2 * r2[buf, c, sl])

            pltpu.sync_copy(ob, out.at[pl.ds(base + s * C, C)])

        fire(0, 0)

        def pair(g, carry):
            s0 = 2 * g
            fire(s0 + 1, 1)
            drain(0)
            compute(s0, 0)

            @pl.when(g + 1 < nch // 2)
            def _():
                fire(s0 + 2, 0)

            drain(1)
            compute(s0 + 1, 1)
            return carry

        lax.fori_loop(0, nch // 2, pair, 0)

    kfn = pl.kernel(
        body,
        out_type=jax.ShapeDtypeStruct((M, D), jnp.float32),
        mesh=mesh,
        compiler_params=pltpu.CompilerParams(needs_layout_passes=False),
        scratch_types=[
            pltpu.VMEM((2, C), jnp.int32),
            pltpu.VMEM((2, C), jnp.int32),
            pltpu.VMEM((2, C), jnp.int32),
            pltpu.VMEM((2, C), jnp.float32),
            pltpu.VMEM((2, C), jnp.float32),
            pltpu.VMEM((2, C), jnp.float32),
            pltpu.VMEM((2, C, D), jnp.float32),
            pltpu.VMEM((2, C, D), jnp.float32),
            pltpu.VMEM((2, C, D), jnp.float32),
            pltpu.VMEM((C, D), jnp.float32),
            pltpu.SemaphoreType.DMA((2, 3)),
        ],
    )
    return kfn(table, i0, i1, i2, w0, w1, w2)


def _mm2_stats(xa, xb, wa, wb, rb=512):
    M, Ka = xa.shape
    Kb = xb.shape[1]
    Cout = wa.shape[0]

    def body(xa_ref, xb_ref, wa_ref, wb_ref, z_ref, s_ref, q_ref):
        z = lax.dot_general(xa_ref[...].astype(jnp.bfloat16),
                            wa_ref[...].astype(jnp.bfloat16),
                            (((1,), (1,)), ((), ())),
                            preferred_element_type=jnp.float32)
        z = z + lax.dot_general(xb_ref[...].astype(jnp.bfloat16),
                                wb_ref[...].astype(jnp.bfloat16),
                                (((1,), (1,)), ((), ())),
                                preferred_element_type=jnp.float32)
        z_ref[...] = z

        @pl.when(pl.program_id(0) == 0)
        def _():
            s_ref[...] = jnp.zeros_like(s_ref)
            q_ref[...] = jnp.zeros_like(q_ref)

        s_ref[...] += jnp.sum(z, axis=0, keepdims=True)
        q_ref[...] += jnp.sum(z * z, axis=0, keepdims=True)

    return pl.pallas_call(
        body,
        grid=(M // rb,),
        in_specs=[
            pl.BlockSpec((rb, Ka), lambda i: (i, 0)),
            pl.BlockSpec((rb, Kb), lambda i: (i, 0)),
            pl.BlockSpec((Cout, Ka), lambda i: (0, 0)),
            pl.BlockSpec((Cout, Kb), lambda i: (0, 0)),
        ],
        out_specs=[
            pl.BlockSpec((rb, Cout), lambda i: (i, 0)),
            pl.BlockSpec((1, Cout), lambda i: (0, 0)),
            pl.BlockSpec((1, Cout), lambda i: (0, 0)),
        ],
        out_shape=[
            jax.ShapeDtypeStruct((M, Cout), jnp.float32),
            jax.ShapeDtypeStruct((1, Cout), jnp.float32),
            jax.ShapeDtypeStruct((1, Cout), jnp.float32),
        ],
    )(xa, xb, wa, wb)


def _bn_scale_shift(s_ref, q_ref, g_ref, b_ref, m):
    mean = s_ref[...] * (1.0 / m)
    var = q_ref[...] * (1.0 / m) - mean * mean
    sc = g_ref[...] * lax.rsqrt(var + 1e-5)
    sh = b_ref[...] - mean * sc
    return sc, sh


def _bn_relu(z, s, q, g, b, rb=512):
    M, C = z.shape

    def body(z_ref, s_ref, q_ref, g_ref, b_ref, y_ref):
        sc, sh = _bn_scale_shift(s_ref, q_ref, g_ref, b_ref, M)
        y_ref[...] = jnp.maximum(z_ref[...] * sc + sh, 0.0)

    return pl.pallas_call(
        body,
        grid=(M // rb,),
        in_specs=[
            pl.BlockSpec((rb, C), lambda i: (i, 0)),
            pl.BlockSpec((1, C), lambda i: (0, 0)),
            pl.BlockSpec((1, C), lambda i: (0, 0)),
            pl.BlockSpec((1, C), lambda i: (0, 0)),
            pl.BlockSpec((1, C), lambda i: (0, 0)),
        ],
        out_specs=pl.BlockSpec((rb, C), lambda i: (i, 0)),
        out_shape=jax.ShapeDtypeStruct((M, C), jnp.float32),
    )(z, s, q, g, b)


def _bn_mm_stats(z, s, q, g, b, w, rb=512):
    M, C = z.shape
    Cout = w.shape[0]

    def body(z_ref, s_ref, q_ref, g_ref, b_ref, w_ref, z2_ref, s2_ref, q2_ref):
        sc, sh = _bn_scale_shift(s_ref, q_ref, g_ref, b_ref, M)
        h = jnp.maximum(z_ref[...] * sc + sh, 0.0)
        z2 = lax.dot_general(h.astype(jnp.bfloat16),
                             w_ref[...].astype(jnp.bfloat16),
                             (((1,), (1,)), ((), ())),
                             preferred_element_type=jnp.float32)
        z2_ref[...] = z2

        @pl.when(pl.program_id(0) == 0)
        def _():
            s2_ref[...] = jnp.zeros_like(s2_ref)
            q2_ref[...] = jnp.zeros_like(q2_ref)

        s2_ref[...] += jnp.sum(z2, axis=0, keepdims=True)
        q2_ref[...] += jnp.sum(z2 * z2, axis=0, keepdims=True)

    return pl.pallas_call(
        body,
        grid=(M // rb,),
        in_specs=[
            pl.BlockSpec((rb, C), lambda i: (i, 0)),
            pl.BlockSpec((1, C), lambda i: (0, 0)),
            pl.BlockSpec((1, C), lambda i: (0, 0)),
            pl.BlockSpec((1, C), lambda i: (0, 0)),
            pl.BlockSpec((1, C), lambda i: (0, 0)),
            pl.BlockSpec((Cout, C), lambda i: (0, 0)),
        ],
        out_specs=[
            pl.BlockSpec((rb, Cout), lambda i: (i, 0)),
            pl.BlockSpec((1, Cout), lambda i: (0, 0)),
            pl.BlockSpec((1, Cout), lambda i: (0, 0)),
        ],
        out_shape=[
            jax.ShapeDtypeStruct((M, Cout), jnp.float32),
            jax.ShapeDtypeStruct((1, Cout), jnp.float32),
            jax.ShapeDtypeStruct((1, Cout), jnp.float32),
        ],
    )(z, s, q, g, b, w)


def _bn_vec(z, s, q, g, b, w, rb=512):
    M, C = z.shape

    def body(z_ref, s_ref, q_ref, g_ref, b_ref, w_ref, o_ref):
        sc, sh = _bn_scale_shift(s_ref, q_ref, g_ref, b_ref, M)
        h = jnp.maximum(z_ref[...] * sc + sh, 0.0)
        o_ref[...] = jnp.sum(h * w_ref[...], axis=1, keepdims=True)

    return pl.pallas_call(
        body,
        grid=(M // rb,),
        in_specs=[
            pl.BlockSpec((rb, C), lambda i: (i, 0)),
            pl.BlockSpec((1, C), lambda i: (0, 0)),
            pl.BlockSpec((1, C), lambda i: (0, 0)),
            pl.BlockSpec((1, C), lambda i: (0, 0)),
            pl.BlockSpec((1, C), lambda i: (0, 0)),
            pl.BlockSpec((1, C), lambda i: (0, 0)),
        ],
        out_specs=pl.BlockSpec((rb, 1), lambda i: (i, 0)),
        out_shape=jax.ShapeDtypeStruct((M, 1), jnp.float32),
    )(z, s, q, g, b, w)


def kernel(xyz_a, xyz_b, xyz_c, feat_a, feat_b, feat_c,
           W_lin0, g0, b0, W_lin1, g1, b1, W_dec0, gd, bd, W_dec1):
    B, Na, _ = xyz_a.shape
    Nb_ = xyz_b.shape[1]
    Sc = xyz_c.shape[1]
    D1, D2, D3 = feat_a.shape[1], feat_b.shape[1], feat_c.shape[1]

    faT = feat_a.transpose(0, 2, 1).reshape(B * Na, D1)
    fbT = feat_b.transpose(0, 2, 1).reshape(B * Nb_, D2)
    fcT = feat_c.transpose(0, 2, 1).reshape(B * Sc, D3)

    i0, i1, i2, w0, w1, w2 = _knn3(xyz_b, xyz_c, 512)
    j0, j1, j2, v0, v1, v2 = _knn3(xyz_a, xyz_b, 1024)

    interp0 = _interp(fcT, i0, i1, i2, w0, w1, w2)
    z0, s0_, q0_ = _mm2_stats(fbT, interp0, W_lin0[:, :D2], W_lin0[:, D2:])
    y0 = _bn_relu(z0, s0_, q0_, g0.reshape(1, -1), b0.reshape(1, -1))

    interp1 = _interp(y0, j0, j1, j2, v0, v1, v2)
    z1, s1_, q1_ = _mm2_stats(faT, interp1, W_lin1[:, :D1], W_lin1[:, D1:])
    z2, s2_, q2_ = _bn_mm_stats(z1, s1_, q1_, g1.reshape(1, -1),
                                b1.reshape(1, -1), W_dec0)
    out = _bn_vec(z2, s2_, q2_, gd.reshape(1, -1), bd.reshape(1, -1), W_dec1)
    return out.reshape(B, Na)
